# Initial kernel scaffold; baseline (speedup 1.0000x reference)
import jax
import jax.numpy as jnp
from jax.experimental import pallas as pl


def kernel(src_node_features, neigh_node_features, src_nodes, weight):
    raise NotImplementedError("write your pallas kernel here")


# speedup vs baseline: 5.4780x; 5.4780x over previous
import functools

import jax
import jax.numpy as jnp
from jax.experimental import pallas as pl

B, N, S, MAXLEN, D, H = 8, 2048, 4, 16, 128, 128
ROWS = B * N
BLK = 256


def _fused_kernel(src_ref, neigh_ref, w_ref, out_ref):
    neigh = neigh_ref[...]
    src = src_ref[...]

    nz_row = jnp.any(neigh != 0.0, axis=-1)
    count = jnp.sum(nz_row.astype(jnp.float32), axis=-1)
    denom = jnp.maximum(count, 1.0)

    agg = jnp.sum(neigh, axis=1) / denom[:, None]
    hidden = jnp.sum(src, axis=1) + agg

    out = jnp.dot(hidden, w_ref[...], preferred_element_type=jnp.float32)
    out_ref[...] = jnp.where(out >= 0.0, out, 0.01 * out)


@jax.jit
def _run(src, neigh, weight):
    src = src.reshape(ROWS, S, D)
    neigh = neigh.reshape(ROWS, MAXLEN, D)
    grid = (ROWS // BLK,)
    return pl.pallas_call(
        _fused_kernel,
        grid=grid,
        in_specs=[
            pl.BlockSpec((BLK, S, D), lambda i: (i, 0, 0)),
            pl.BlockSpec((BLK, MAXLEN, D), lambda i: (i, 0, 0)),
            pl.BlockSpec((D, H), lambda i: (0, 0)),
        ],
        out_specs=pl.BlockSpec((BLK, H), lambda i: (i, 0)),
        out_shape=jax.ShapeDtypeStruct((ROWS, H), jnp.float32),
    )(src, neigh, weight)


def kernel(src_node_features, neigh_node_features, src_nodes, weight):
    return _run(src_node_features, neigh_node_features, weight)
